# Initial kernel scaffold; baseline (speedup 1.0000x reference)
import jax
import jax.numpy as jnp
from jax.experimental import pallas as pl


def kernel(x, edge_index, W1, b1, W2, b2):
    raise NotImplementedError("write your pallas kernel here")


# speedup vs baseline: 24.3022x; 24.3022x over previous
import functools

import jax
import jax.numpy as jnp
from jax import lax
from jax.experimental import pallas as pl
from jax.experimental.pallas import tpu as pltpu
from jax.experimental.pallas import tpu_sc as plsc

F = 16
SPC = 16
NC = 2
NW = NC * SPC
CHUNK = 128


def _propagate(hs, sidx, didx, np_, ch_w):
  rps = np_ // SPC

  mesh = plsc.VectorSubcoreMesh(core_axis_name="c", subcore_axis_name="s")

  @functools.partial(
      pl.kernel,
      mesh=mesh,
      out_type=[
          jax.ShapeDtypeStruct((np_, F), jnp.float32),
          jax.ShapeDtypeStruct((np_, F), jnp.float32),
      ],
      scratch_types=[
          pltpu.VMEM_SHARED((np_, F), jnp.float32),
          pltpu.VMEM((ch_w, CHUNK), jnp.int32),
          pltpu.VMEM((ch_w, CHUNK), jnp.int32),
          pltpu.VMEM((CHUNK, F), jnp.float32),
          pltpu.SemaphoreType.DMA,
      ],
      compiler_params=pltpu.CompilerParams(use_tc_tiling_on_sc=False),
  )
  def k(hs_hbm, sidx_hbm, didx_hbm, pa_hbm, pb_hbm, acc, sv, dv, rows, sem):
    c = lax.axis_index("c")
    s = lax.axis_index("s")
    w = c * SPC + s
    r0 = s * rps
    pltpu.sync_copy(hs_hbm.at[pl.ds(r0, rps)], acc.at[pl.ds(r0, rps)])
    pltpu.sync_copy(sidx_hbm.at[pl.ds(w * ch_w, ch_w)], sv)
    pltpu.sync_copy(didx_hbm.at[pl.ds(w * ch_w, ch_w)], dv)
    plsc.subcore_barrier()

    def body(j, carry):
      pltpu.async_copy(hs_hbm.at[sv.at[j]], rows, sem).wait()
      pltpu.sync_copy(rows, acc.at[dv.at[j]], add=True)
      return carry

    lax.fori_loop(0, ch_w, body, 0)

    plsc.subcore_barrier()

    @pl.when(c == 0)
    def _():
      pltpu.sync_copy(acc.at[pl.ds(r0, rps)], pa_hbm.at[pl.ds(r0, rps)])

    @pl.when(c == 1)
    def _():
      pltpu.sync_copy(acc.at[pl.ds(r0, rps)], pb_hbm.at[pl.ds(r0, rps)])

  return k(hs, sidx, didx)


def _tc_prep(da, db, x, w1, np_):
  def body(da_ref, db_ref, x_ref, w1_ref, dinv_ref, hs_ref):
    deg = da_ref[:, 0:1] + db_ref[:, 0:1] - 1.0
    dinv = lax.rsqrt(deg)
    h = jnp.dot(x_ref[...], w1_ref[...], preferred_element_type=jnp.float32)
    dinv_ref[...] = dinv
    hs_ref[...] = dinv * h

  return pl.pallas_call(
      body,
      out_shape=[
          jax.ShapeDtypeStruct((np_, 1), jnp.float32),
          jax.ShapeDtypeStruct((np_, F), jnp.float32),
      ],
  )(da, db, x, w1)


def _tc_layer(dinv, pa, pb, hsp, b1, w2p, np_):
  def body(dinv_ref, pa_ref, pb_ref, hsp_ref, b1_ref, w2_ref, out_ref):
    t = dinv_ref[...] * (pa_ref[...] + pb_ref[...] - hsp_ref[...]) + b1_ref[...]
    h2 = jnp.maximum(t, 0.0)
    out_ref[...] = dinv_ref[...] * jnp.dot(
        h2, w2_ref[...], preferred_element_type=jnp.float32)

  return pl.pallas_call(
      body,
      out_shape=jax.ShapeDtypeStruct((np_, F), jnp.float32),
  )(dinv, pa, pb, hsp, b1, w2p)


def _tc_final(dinv, pa, pb, hsp, b2p, np_, c_):
  def body(dinv_ref, pa_ref, pb_ref, hsp_ref, b2_ref, out_ref):
    t = dinv_ref[...] * (pa_ref[...] + pb_ref[...] - hsp_ref[...]) + b2_ref[...]
    r = jnp.maximum(t, 0.0)
    col = lax.broadcasted_iota(jnp.int32, (np_, F), 1)
    valid = col < c_
    rm = jnp.where(valid, r, jnp.float32(-1e30))
    m = jnp.max(rm, axis=1, keepdims=True)
    e = jnp.where(valid, jnp.exp(rm - m), 0.0)
    ssum = jnp.sum(e, axis=1, keepdims=True)
    out_ref[...] = rm - m - jnp.log(ssum)

  return pl.pallas_call(
      body,
      out_shape=jax.ShapeDtypeStruct((np_, F), jnp.float32),
  )(dinv, pa, pb, hsp, b2p)


def kernel(x, edge_index, W1, b1, W2, b2):
  n, d = x.shape
  h = W1.shape[1]
  c_ = W2.shape[1]
  assert h == F
  e = edge_index.shape[1]

  np_ = ((n + 255) // 256) * 256
  ch_w = -(-e // (NW * CHUNK))
  ch_w = ((ch_w + 7) // 8) * 8
  ep = NW * ch_w * CHUNK

  src = edge_index[0]
  dst = edge_index[1]
  pad = jnp.full((ep - e,), n, dtype=jnp.int32)
  sidx = jnp.concatenate([src, pad]).reshape(NW * ch_w, CHUNK)
  didx = jnp.concatenate([dst, pad]).reshape(NW * ch_w, CHUNK)

  xp = jnp.pad(x, ((0, np_ - n), (0, 0)))
  ones = jnp.ones((np_, F), jnp.float32)

  da, db = _propagate(ones, sidx, didx, np_, ch_w)
  dinv, hs1 = _tc_prep(da, db, xp, W1, np_)
  pa, pb = _propagate(hs1, sidx, didx, np_, ch_w)
  w2p = jnp.pad(W2, ((0, 0), (0, F - c_)))
  hs2 = _tc_layer(dinv, pa, pb, hs1, b1.reshape(1, F), w2p, np_)
  qa, qb = _propagate(hs2, sidx, didx, np_, ch_w)
  b2p = jnp.pad(b2, (0, F - c_)).reshape(1, F)
  out = _tc_final(dinv, qa, qb, hs2, b2p, np_, c_)
  return out[:n, :c_]


# speedup vs baseline: 33.7472x; 1.3886x over previous
import functools

import jax
import jax.numpy as jnp
from jax import lax
from jax.experimental import pallas as pl
from jax.experimental.pallas import tpu as pltpu
from jax.experimental.pallas import tpu_sc as plsc

F = 16
SPC = 16
NC = 2
NW = NC * SPC
CHUNK = 128


def _propagate(hs, sidx, didx, np_, ch_w):
  rps = np_ // SPC

  mesh = plsc.VectorSubcoreMesh(core_axis_name="c", subcore_axis_name="s")

  @functools.partial(
      pl.kernel,
      mesh=mesh,
      out_type=[
          jax.ShapeDtypeStruct((np_, F), jnp.float32),
          jax.ShapeDtypeStruct((np_, F), jnp.float32),
      ],
      scratch_types=[
          pltpu.VMEM_SHARED((np_, F), jnp.float32),
          pltpu.VMEM((ch_w, CHUNK), jnp.int32),
          pltpu.VMEM((ch_w, CHUNK), jnp.int32),
          pltpu.VMEM((2, CHUNK, F), jnp.float32),
          pltpu.SemaphoreType.DMA,
      ],
      compiler_params=pltpu.CompilerParams(use_tc_tiling_on_sc=False),
  )
  def k(hs_hbm, sidx_hbm, didx_hbm, pa_hbm, pb_hbm, acc, sv, dv, rows, sem):
    c = lax.axis_index("c")
    s = lax.axis_index("s")
    w = c * SPC + s
    r0 = s * rps
    pltpu.sync_copy(hs_hbm.at[pl.ds(r0, rps)], acc.at[pl.ds(r0, rps)])
    pltpu.sync_copy(sidx_hbm.at[pl.ds(w * ch_w, ch_w)], sv)
    pltpu.sync_copy(didx_hbm.at[pl.ds(w * ch_w, ch_w)], dv)
    plsc.subcore_barrier()

    pltpu.async_copy(hs_hbm.at[sv.at[0]], rows.at[0], sem)

    def body(j, carry):
      cur = lax.rem(j, 2)
      pltpu.make_async_copy(hs_hbm.at[sv.at[j]], rows.at[cur], sem).wait()

      @pl.when(j + 1 < ch_w)
      def _():
        pltpu.async_copy(hs_hbm.at[sv.at[j + 1]], rows.at[1 - cur], sem)

      pltpu.sync_copy(rows.at[cur], acc.at[dv.at[j]], add=True)
      return carry

    lax.fori_loop(0, ch_w, body, 0)

    plsc.subcore_barrier()

    @pl.when(c == 0)
    def _():
      pltpu.sync_copy(acc.at[pl.ds(r0, rps)], pa_hbm.at[pl.ds(r0, rps)])

    @pl.when(c == 1)
    def _():
      pltpu.sync_copy(acc.at[pl.ds(r0, rps)], pb_hbm.at[pl.ds(r0, rps)])

  return k(hs, sidx, didx)


def _degrees(didx, np_, ch_w):
  rps = np_ // SPC

  mesh = plsc.VectorSubcoreMesh(core_axis_name="c", subcore_axis_name="s")

  @functools.partial(
      pl.kernel,
      mesh=mesh,
      out_type=[
          jax.ShapeDtypeStruct((np_,), jnp.float32),
          jax.ShapeDtypeStruct((np_,), jnp.float32),
      ],
      scratch_types=[
          pltpu.VMEM_SHARED((np_,), jnp.float32),
          pltpu.VMEM((ch_w, CHUNK), jnp.int32),
          pltpu.VMEM((rps,), jnp.float32),
      ],
      compiler_params=pltpu.CompilerParams(use_tc_tiling_on_sc=False),
  )
  def k(didx_hbm, da_hbm, db_hbm, accd, dv, buf):
    c = lax.axis_index("c")
    s = lax.axis_index("s")
    w = c * SPC + s
    r0 = s * rps
    for i in range(rps // F):
      buf[pl.ds(i * F, F)] = jnp.full((F,), 1.0, jnp.float32)
    pltpu.sync_copy(buf, accd.at[pl.ds(r0, rps)])
    pltpu.sync_copy(didx_hbm.at[pl.ds(w * ch_w, ch_w)], dv)
    plsc.subcore_barrier()

    def body(j, carry):
      pltpu.sync_copy(buf.at[pl.ds(0, CHUNK)], accd.at[dv.at[j]], add=True)
      return carry

    lax.fori_loop(0, ch_w, body, 0)

    plsc.subcore_barrier()

    @pl.when(c == 0)
    def _():
      pltpu.sync_copy(accd.at[pl.ds(r0, rps)], da_hbm.at[pl.ds(r0, rps)])

    @pl.when(c == 1)
    def _():
      pltpu.sync_copy(accd.at[pl.ds(r0, rps)], db_hbm.at[pl.ds(r0, rps)])

  return k(didx)


def _tc_prep(da, db, x, w1, np_):
  def body(da_ref, db_ref, x_ref, w1_ref, dinv_ref, hs_ref):
    deg = da_ref[:, 0:1] + db_ref[:, 0:1] - 1.0
    dinv = lax.rsqrt(deg)
    h = jnp.dot(x_ref[...], w1_ref[...], preferred_element_type=jnp.float32)
    dinv_ref[...] = dinv
    hs_ref[...] = dinv * h

  return pl.pallas_call(
      body,
      out_shape=[
          jax.ShapeDtypeStruct((np_, 1), jnp.float32),
          jax.ShapeDtypeStruct((np_, F), jnp.float32),
      ],
  )(da, db, x, w1)


def _tc_layer(dinv, pa, pb, hsp, b1, w2p, np_):
  def body(dinv_ref, pa_ref, pb_ref, hsp_ref, b1_ref, w2_ref, out_ref):
    t = dinv_ref[...] * (pa_ref[...] + pb_ref[...] - hsp_ref[...]) + b1_ref[...]
    h2 = jnp.maximum(t, 0.0)
    out_ref[...] = dinv_ref[...] * jnp.dot(
        h2, w2_ref[...], preferred_element_type=jnp.float32)

  return pl.pallas_call(
      body,
      out_shape=jax.ShapeDtypeStruct((np_, F), jnp.float32),
  )(dinv, pa, pb, hsp, b1, w2p)


def _tc_final(dinv, pa, pb, hsp, b2p, np_, c_):
  def body(dinv_ref, pa_ref, pb_ref, hsp_ref, b2_ref, out_ref):
    t = dinv_ref[...] * (pa_ref[...] + pb_ref[...] - hsp_ref[...]) + b2_ref[...]
    r = jnp.maximum(t, 0.0)
    col = lax.broadcasted_iota(jnp.int32, (np_, F), 1)
    valid = col < c_
    rm = jnp.where(valid, r, jnp.float32(-1e30))
    m = jnp.max(rm, axis=1, keepdims=True)
    e = jnp.where(valid, jnp.exp(rm - m), 0.0)
    ssum = jnp.sum(e, axis=1, keepdims=True)
    out_ref[...] = rm - m - jnp.log(ssum)

  return pl.pallas_call(
      body,
      out_shape=jax.ShapeDtypeStruct((np_, F), jnp.float32),
  )(dinv, pa, pb, hsp, b2p)


def kernel(x, edge_index, W1, b1, W2, b2):
  n, d = x.shape
  h = W1.shape[1]
  c_ = W2.shape[1]
  assert h == F
  e = edge_index.shape[1]

  np_ = ((n + 255) // 256) * 256
  ch_w = -(-e // (NW * CHUNK))
  ch_w = ((ch_w + 7) // 8) * 8
  ep = NW * ch_w * CHUNK

  src = edge_index[0]
  dst = edge_index[1]
  pad = jnp.full((ep - e,), n, dtype=jnp.int32)
  sidx = jnp.concatenate([src, pad]).reshape(NW * ch_w, CHUNK)
  didx = jnp.concatenate([dst, pad]).reshape(NW * ch_w, CHUNK)

  xp = jnp.pad(x, ((0, np_ - n), (0, 0)))

  da, db = _degrees(didx, np_, ch_w)
  dinv, hs1 = _tc_prep(da.reshape(np_, 1), db.reshape(np_, 1), xp, W1, np_)
  pa, pb = _propagate(hs1, sidx, didx, np_, ch_w)
  w2p = jnp.pad(W2, ((0, 0), (0, F - c_)))
  hs2 = _tc_layer(dinv, pa, pb, hs1, b1.reshape(1, F), w2p, np_)
  qa, qb = _propagate(hs2, sidx, didx, np_, ch_w)
  b2p = jnp.pad(b2, (0, F - c_)).reshape(1, F)
  out = _tc_final(dinv, qa, qb, hs2, b2p, np_, c_)
  return out[:n, :c_]
